# baseline (speedup 1.0000x reference)
import functools

import jax
import jax.numpy as jnp
from jax import lax
from jax.experimental import pallas as pl
from jax.experimental.pallas import tpu as pltpu
from jax.experimental.pallas import tpu_sc as plsc

VOCAB = 1000
VOCAB_PAD = 1024
BATCH = 1024
SEQ = 200
N_TOKENS = BATCH * SEQ
NUM_CORES = 2
NUM_SUBCORES = 16
NUM_WORKERS = NUM_CORES * NUM_SUBCORES
B_PER_W = N_TOKENS // NUM_WORKERS
NSTRIP = VOCAB_PAD // 128
NBLK = B_PER_W // 128
LAST_ROWS = VOCAB - 128 * (NSTRIP - 1)


@functools.partial(
    pl.kernel,
    out_type=jax.ShapeDtypeStruct((SEQ * VOCAB, BATCH), jnp.float32),
    mesh=plsc.VectorSubcoreMesh(core_axis_name="c", subcore_axis_name="s"),
    compiler_params=pltpu.CompilerParams(needs_layout_passes=False),
    scratch_types=[
        pltpu.VMEM((B_PER_W,), jnp.int32),
        [pltpu.VMEM((NSTRIP, 128), jnp.int32) for _ in range(2)],
        [pltpu.VMEM((128, 128), jnp.float32) for _ in range(2)],
        [pltpu.VMEM((128, 128), jnp.float32) for _ in range(2)],
        [pltpu.SemaphoreType.DMA for _ in range(2)],
        [pltpu.SemaphoreType.DMA for _ in range(2)],
    ],
)
def _gather_t_kernel(idx_hbm, table_hbm, out_hbm, idx_v, idx_s, rows_v, t_v,
                     sem_g, sem_w):
    wid = lax.axis_index("s") * NUM_CORES + lax.axis_index("c")
    base = wid * B_PER_W
    pltpu.sync_copy(idx_hbm.at[pl.ds(base, B_PER_W)], idx_v)

    vvecs = [lax.iota(jnp.int32, 16) + 16 * v0 for v0 in range(8)]

    def fill_idx_s(q, bl):
        def row(t8, c):
            for k in range(NSTRIP):
                for j in range(2):
                    chunk = idx_v[pl.ds(128 * bl + 16 * (2 * t8 + j), 16)]
                    idx_s[q][k, pl.ds(16 * (2 * t8 + j), 16)] = chunk + 1000 * k
            return c
        lax.fori_loop(0, 4, row, 0)

    def start_gather(q, k, p):
        pltpu.async_copy(table_hbm.at[idx_s[q].at[k]], rows_v[p], sem_g[p])

    def wait_gather(p):
        pltpu.make_async_copy(table_hbm.at[idx_s[0].at[0]], rows_v[p],
                              sem_g[p]).wait()

    def start_write(p, m0, col, nrows):
        pltpu.async_copy(t_v[p].at[pl.ds(0, nrows)],
                         out_hbm.at[pl.ds(m0, nrows), pl.ds(col, 128)],
                         sem_w[p])

    def wait_write(p, nrows):
        pltpu.make_async_copy(t_v[p].at[pl.ds(0, nrows)],
                              out_hbm.at[pl.ds(0, nrows), pl.ds(0, 128)],
                              sem_w[p]).wait()

    def transpose(p):
        def trow(vg, c):
            vbase = vg * 4
            vg_vec = jnp.full((16,), vbase, jnp.int32)
            for dv in range(4):
                vvec = vg_vec + dv
                v = vbase + dv
                for r0 in range(8):
                    x = plsc.load_gather(rows_v[p], [vvecs[r0], vvec])
                    t_v[p][v, pl.ds(16 * r0, 16)] = x
            return c
        lax.fori_loop(0, 32, trow, 0)

    fill_idx_s(0, 0)
    start_gather(0, 0, 0)
    start_gather(0, 1, 1)

    def body(g, c):
        for half in range(2):
            bl = 2 * g + half
            q, nq = half, 1 - half
            beta = wid * NBLK + bl
            s = beta // NSTRIP
            j = beta % NSTRIP
            col = 128 * j

            for k in range(NSTRIP):
                p = k % 2
                nrows = LAST_ROWS if k == NSTRIP - 1 else 128
                prev_rows = LAST_ROWS if (k - 2) % NSTRIP == NSTRIP - 1 else 128
                if k >= 2 or half == 1:
                    wait_write(p, prev_rows)
                else:
                    @pl.when(bl > 0)
                    def _(p=p, prev_rows=prev_rows):
                        wait_write(p, prev_rows)
                wait_gather(p)
                if k == 5:
                    @pl.when(bl + 1 < NBLK)
                    def _(nq=nq, bl=bl):
                        fill_idx_s(nq, bl + 1)
                transpose(p)
                start_write(p, s * VOCAB + 128 * k, col, nrows)
                if k < 6:
                    start_gather(q, k + 2, p)
                else:
                    @pl.when(bl + 1 < NBLK)
                    def _(nq=nq, k=k, p=p):
                        start_gather(nq, k - 6, p)
        return c

    lax.fori_loop(0, NBLK // 2, body, 0)
    wait_write(0, 128)
    wait_write(1, LAST_ROWS)


def kernel(inputs, embedding):
    idx2 = inputs.T.reshape(-1).astype(jnp.int32)
    table = jnp.pad(embedding, ((0, 0), (0, VOCAB_PAD - VOCAB)))
    t3 = table.reshape(VOCAB, NSTRIP, 128).transpose(1, 0, 2).reshape(-1, 128)
    out2d = _gather_t_kernel(idx2, t3)
    return out2d.reshape(SEQ, VOCAB, BATCH).transpose(2, 0, 1)


# speedup vs baseline: 4.0546x; 4.0546x over previous
import functools

import jax
import jax.numpy as jnp
from jax import lax
from jax.experimental import pallas as pl
from jax.experimental.pallas import tpu as pltpu
from jax.experimental.pallas import tpu_sc as plsc

VOCAB = 1000
VOCAB_PAD = 1024
TAIL_START = 896
TAIL = VOCAB - TAIL_START
N_TOKENS = 1024 * 200
NUM_CORES = 2
NUM_SUBCORES = 16
NUM_WORKERS = NUM_CORES * NUM_SUBCORES
B_PER_W = N_TOKENS // NUM_WORKERS
CHUNK = 16
N_CHUNKS = B_PER_W // CHUNK
NBUF = 4


@functools.partial(
    pl.kernel,
    out_type=jax.ShapeDtypeStruct((N_TOKENS, VOCAB), jnp.float32),
    mesh=plsc.VectorSubcoreMesh(core_axis_name="c", subcore_axis_name="s"),
    scratch_types=[
        pltpu.VMEM((B_PER_W,), jnp.int32),
        [pltpu.VMEM((CHUNK, VOCAB_PAD), jnp.float32) for _ in range(NBUF)],
        [pltpu.VMEM((CHUNK, TAIL), jnp.float32) for _ in range(NBUF)],
        [pltpu.SemaphoreType.DMA for _ in range(NBUF)],
        [pltpu.SemaphoreType.DMA for _ in range(NBUF)],
        [pltpu.SemaphoreType.DMA for _ in range(NBUF)],
    ],
)
def _gather_kernel(idx_hbm, table_hbm, out_hbm, idx_v, rows_v, tail_v,
                   sem_g, sem_b, sem_t):
    wid = lax.axis_index("s") * NUM_CORES + lax.axis_index("c")
    base = wid * B_PER_W

    pltpu.sync_copy(idx_hbm.at[pl.ds(base, B_PER_W)], idx_v)

    def start_gather(i, b):
        pltpu.async_copy(
            table_hbm.at[idx_v.at[pl.ds(i * CHUNK, CHUNK)]], rows_v[b],
            sem_g[b])

    def finish_chunk(i, b):
        pltpu.make_async_copy(
            table_hbm.at[idx_v.at[pl.ds(i * CHUNK, CHUNK)]], rows_v[b],
            sem_g[b]).wait()
        off = base + i * CHUNK
        pltpu.async_copy(
            rows_v[b].at[:, pl.ds(0, TAIL_START)],
            out_hbm.at[pl.ds(off, CHUNK), pl.ds(0, TAIL_START)], sem_b[b])

        def repack_row(r, c):
            for t in range(6):
                tail_v[b][r, pl.ds(16 * t, 16)] = (
                    rows_v[b][r, pl.ds(TAIL_START + 16 * t, 16)])
            tail_v[b][r, pl.ds(TAIL - 16, 16)] = (
                rows_v[b][r, pl.ds(VOCAB - 16, 16)])
            return c

        lax.fori_loop(0, CHUNK, repack_row, 0)
        pltpu.async_copy(
            tail_v[b],
            out_hbm.at[pl.ds(off, CHUNK), pl.ds(TAIL_START, TAIL)], sem_t[b])

    def wait_out(i, b):
        off = base + i * CHUNK
        pltpu.make_async_copy(
            rows_v[b].at[:, pl.ds(0, TAIL_START)],
            out_hbm.at[pl.ds(off, CHUNK), pl.ds(0, TAIL_START)],
            sem_b[b]).wait()
        pltpu.make_async_copy(
            tail_v[b],
            out_hbm.at[pl.ds(off, CHUNK), pl.ds(TAIL_START, TAIL)],
            sem_t[b]).wait()

    start_gather(0, 0)

    def body(g, c):
        for b in range(NBUF):
            i = g * NBUF + b

            @pl.when(i + 1 < N_CHUNKS)
            def _(i=i, nb=(b + 1) % NBUF):
                @pl.when(i >= NBUF - 1)
                def _():
                    wait_out(i + 1 - NBUF, nb)
                start_gather(i + 1, nb)

            finish_chunk(i, b)
        return c

    lax.fori_loop(0, N_CHUNKS // NBUF, body, 0)
    for c in range(N_CHUNKS - NBUF, N_CHUNKS):
        wait_out(c, c % NBUF)


def kernel(inputs, embedding):
    idx = inputs.reshape(-1).astype(jnp.int32)
    table = jnp.pad(embedding, ((0, 0), (0, VOCAB_PAD - VOCAB)))
    out = _gather_kernel(idx, table)
    return out.reshape(inputs.shape[0], inputs.shape[1], VOCAB)
